# baseline (speedup 1.0000x reference)
import jax
import jax.numpy as jnp
from jax.experimental import pallas as pl
from jax.experimental.pallas import tpu as pltpu

_RES = 32
_TP = 100
_EXPAND = 0.5
_CH = 20
_NL = _CH * _RES
_NEG = -1e9


def _prop_body(sc_ref, er_ref, ac_ref, ar_ref, gt_ref,
               si_ref, ei_ref, iou_ref, lo_ref, hi_ref, w_ref,
               score_scr, rowmax_scr):
    sc = sc_ref[0]
    ac = ac_ref[0]
    er = er_ref[0]
    ar = ar_ref[0]
    sub = jax.lax.broadcasted_iota(jnp.int32, (128, 128), 0)
    lane = jax.lax.broadcasted_iota(jnp.int32, (128, 128), 1)
    score = (sc * er) * 0.5 * (ac + ar)
    valid = (lane > sub) & (lane < _TP)
    score_scr[...] = jnp.where(valid, score, _NEG)
    rowmax_scr[...] = jnp.max(score_scr[...], axis=1, keepdims=True)

    colio = jax.lax.broadcasted_iota(jnp.int32, (128, 1), 0)
    lane1 = jax.lax.broadcasted_iota(jnp.int32, (1, 128), 1)
    pts = (jax.lax.broadcasted_iota(jnp.int32, (1, _RES), 1).astype(jnp.float32)
           + 0.5) / _RES

    def body(k, carry):
        si_acc, ei_acc, iou_acc = carry
        rm = rowmax_scr[...]
        gmax = jnp.max(rm)
        r = jnp.min(jnp.where(rm == gmax, colio, 127))
        row = score_scr[pl.ds(r, 1), :]
        e = jnp.min(jnp.where(row == gmax, lane1, 127))
        newrow = jnp.where(lane1 == e, 2.0 * _NEG, row)
        score_scr[pl.ds(r, 1), :] = newrow
        rowmax_scr[pl.ds(r, 1), :] = jnp.max(newrow, axis=1, keepdims=True)
        si_acc = jnp.where(lane1 == k, r, si_acc)
        ei_acc = jnp.where(lane1 == k, e, ei_acc)
        gtrow = gt_ref[0, pl.ds(r, 1), :]
        iouk = jnp.sum(jnp.where(lane1 == e, gtrow, 0.0))
        iou_acc = jnp.where(lane1 == k, iouk, iou_acc)
        s_f = r.astype(jnp.float32)
        e_f = e.astype(jnp.float32)
        ctx = (e_f - s_f) * _EXPAND
        s_exp = s_f - ctx
        e_exp = e_f + ctx
        coords = s_exp + (e_exp - s_exp) * pts
        coords = jnp.clip(coords, 0.0, _TP - 1.0)
        lo = jnp.floor(coords).astype(jnp.int32)
        hi = jnp.minimum(lo + 1, _TP - 1)
        ww = coords - lo.astype(jnp.float32)
        lo_ref[0, pl.ds(k, 1), :] = lo
        hi_ref[0, pl.ds(k, 1), :] = hi
        w_ref[0, pl.ds(k, 1), :] = ww
        return si_acc, ei_acc, iou_acc

    zi = jnp.zeros((1, 128), jnp.int32)
    zf = jnp.zeros((1, 128), jnp.float32)
    si_acc, ei_acc, iou_acc = jax.lax.fori_loop(0, _TP, body, (zi, zi, zf))
    si_ref[0] = si_acc
    ei_ref[0] = ei_acc
    iou_ref[0] = iou_acc


def _proposals(start, end, actionnes, gt_iou_map):
    B, T = start.shape
    sc = jnp.pad(start[:, :, None], ((0, 0), (0, 128 - T), (0, 0)))
    er = jnp.pad(end[:, None, :], ((0, 0), (0, 0), (0, 128 - T)))
    ac = jnp.pad(actionnes[:, :, None], ((0, 0), (0, 128 - T), (0, 0)))
    ar = jnp.pad(actionnes[:, None, :], ((0, 0), (0, 0), (0, 128 - T)))
    gt = jnp.pad(gt_iou_map, ((0, 0), (0, 104 - T), (0, 128 - T)))
    bmap = lambda b: (b, 0, 0)
    return pl.pallas_call(
        _prop_body,
        grid=(B,),
        in_specs=[
            pl.BlockSpec((1, 128, 1), bmap),
            pl.BlockSpec((1, 1, 128), bmap),
            pl.BlockSpec((1, 128, 1), bmap),
            pl.BlockSpec((1, 1, 128), bmap),
            pl.BlockSpec((1, 104, 128), bmap),
        ],
        out_specs=[
            pl.BlockSpec((1, 1, 128), bmap),
            pl.BlockSpec((1, 1, 128), bmap),
            pl.BlockSpec((1, 1, 128), bmap),
            pl.BlockSpec((1, 128, _RES), bmap),
            pl.BlockSpec((1, 128, _RES), bmap),
            pl.BlockSpec((1, 128, _RES), bmap),
        ],
        out_shape=[
            jax.ShapeDtypeStruct((B, 1, 128), jnp.int32),
            jax.ShapeDtypeStruct((B, 1, 128), jnp.int32),
            jax.ShapeDtypeStruct((B, 1, 128), jnp.float32),
            jax.ShapeDtypeStruct((B, 128, _RES), jnp.int32),
            jax.ShapeDtypeStruct((B, 128, _RES), jnp.int32),
            jax.ShapeDtypeStruct((B, 128, _RES), jnp.float32),
        ],
        scratch_shapes=[
            pltpu.VMEM((128, 128), jnp.float32),
            pltpu.VMEM((128, 1), jnp.float32),
        ],
        compiler_params=pltpu.CompilerParams(
            dimension_semantics=("parallel",)),
    )(sc, er, ac, ar, gt)


def _align_body(lo_ref, hi_ref, w_ref, x_ref, out_ref):
    T = x_ref.shape[2]
    lo = jnp.broadcast_to(lo_ref[0, 0], (T, _NL))
    hi = jnp.broadcast_to(hi_ref[0, 0], (T, _NL))
    w = jnp.broadcast_to(w_ref[0, 0], (T, _NL))
    t = jax.lax.broadcasted_iota(jnp.int32, (T, _NL), 0)
    wmat = jnp.where(t == lo, 1.0 - w, 0.0) + jnp.where(t == hi, w, 0.0)
    res = jax.lax.dot_general(x_ref[0], wmat, (((1,), (0,)), ((), ())),
                              preferred_element_type=jnp.float32)
    for i in range(_CH):
        out_ref[i] = res[:, i * _RES:(i + 1) * _RES]


def _align(x, lo_r, hi_r, w_r):
    B, C, T = x.shape
    nch = _TP // _CH
    spec_idx = pl.BlockSpec((1, 1, 1, _NL), lambda b, c: (b, c, 0, 0))
    return pl.pallas_call(
        _align_body,
        grid=(B, nch),
        in_specs=[
            spec_idx, spec_idx, spec_idx,
            pl.BlockSpec((1, C, T), lambda b, c: (b, 0, 0)),
        ],
        out_specs=pl.BlockSpec((_CH, C, _RES), lambda b, c: (b * nch + c, 0, 0)),
        out_shape=jax.ShapeDtypeStruct((B * _TP, C, _RES), jnp.float32),
        compiler_params=pltpu.CompilerParams(
            dimension_semantics=("parallel", "arbitrary")),
    )(lo_r, hi_r, w_r, x)


def kernel(x, start, end, actionnes, gt_iou_map, gt_bbox, num_gt):
    B, C, T = x.shape
    nch = _TP // _CH
    si_o, ei_o, iou_o, lo_o, hi_o, w_o = _proposals(
        start, end, actionnes, gt_iou_map)
    s_i = si_o[:, 0, :_TP].reshape(-1)
    e_i = ei_o[:, 0, :_TP].reshape(-1)
    b_idx = jnp.repeat(jnp.arange(B, dtype=jnp.int32), _TP)
    s_f = s_i.astype(jnp.float32)
    e_f = e_i.astype(jnp.float32)
    anchor_coord = jnp.stack([b_idx.astype(jnp.float32), s_f, e_f], axis=1)
    iou = iou_o[:, 0, :_TP].reshape(-1)
    samp_thr = jnp.mean(iou)
    pos_idx_st_end = (iou > samp_thr).astype(jnp.float32)
    lo_r = lo_o[:, :_TP, :].reshape(B, nch, 1, _NL)
    hi_r = hi_o[:, :_TP, :].reshape(B, nch, 1, _NL)
    w_r = w_o[:, :_TP, :].reshape(B, nch, 1, _NL)
    feat = _align(x, lo_r, hi_r, w_r)
    anchor_num = jnp.full((B,), _TP, dtype=jnp.int32)
    return feat, anchor_coord, anchor_num, samp_thr, pos_idx_st_end


# speedup vs baseline: 1.1817x; 1.1817x over previous
import jax
import jax.numpy as jnp
from jax.experimental import pallas as pl
from jax.experimental.pallas import tpu as pltpu

_RES = 32
_TP = 100
_EXPAND = 0.5
_CH = 20
_NL = _CH * _RES
_NEG = -1e9


def _prop_body(sc_ref, er_ref, ac_ref, ar_ref, gt_ref,
               si_ref, ei_ref, iou_ref, lo_ref, hi_ref, w_ref,
               score_scr):
    sc = sc_ref[0]
    ac = ac_ref[0]
    er = er_ref[0]
    ar = ar_ref[0]
    sub = jax.lax.broadcasted_iota(jnp.int32, (128, 128), 0)
    lane = jax.lax.broadcasted_iota(jnp.int32, (128, 128), 1)
    subc = jax.lax.broadcasted_iota(jnp.int32, (128, 1), 0)
    lane1 = jax.lax.broadcasted_iota(jnp.int32, (1, 128), 1)
    score = (sc * er) * 0.5 * (ac + ar)
    valid = (lane > sub) & (lane < _TP)
    score = jnp.where(valid, score, _NEG)
    score_scr[...] = score
    rm0 = jnp.max(score, axis=1, keepdims=True)
    ra0 = jnp.min(jnp.where(score == rm0, lane, 999),
                  axis=1, keepdims=True)

    def body(k, carry):
        rm, ra, si_acc, ei_acc = carry
        gmax = jnp.max(rm)
        r = jnp.min(jnp.where(rm == gmax, subc, 127))
        rsel = subc == r
        e = jnp.sum(jnp.where(rsel, ra, 0))
        si_acc = jnp.where(lane1 == k, r, si_acc)
        ei_acc = jnp.where(lane1 == k, e, ei_acc)
        rowv = jnp.where(rsel, 1.0, 0.0)
        colv = jnp.where(lane1 == e, 1.0, 0.0)
        rows = score_scr[pl.ds(r, 1), :]
        score_scr[...] = score_scr[...] - (4e9 * rowv) * colv
        rows2 = rows - 4e9 * colv
        m1 = jnp.max(rows2)
        a1 = jnp.min(jnp.where(rows2 == m1, lane1, 999))
        rm = jnp.where(rsel, m1, rm)
        ra = jnp.where(rsel, a1, ra)
        return rm, ra, si_acc, ei_acc

    zi = jnp.zeros((1, 128), jnp.int32)
    rm, ra, si_acc, ei_acc = jax.lax.fori_loop(0, _TP, body,
                                               (rm0, ra0, zi, zi))
    si_ref[0] = si_acc
    ei_ref[0] = ei_acc
    hp = jax.lax.Precision.HIGHEST
    eye = jnp.where(sub == lane, 1.0, 0.0)
    si_f = si_acc.astype(jnp.float32)
    ei_f = ei_acc.astype(jnp.float32)
    si_col = jax.lax.dot_general(eye, si_f, (((1,), (1,)), ((), ())),
                                 precision=hp)
    ei_col = jax.lax.dot_general(eye, ei_f, (((1,), (1,)), ((), ())),
                                 precision=hp)
    lane_f = lane.astype(jnp.float32)
    ohs = jnp.where(lane_f == si_col, 1.0, 0.0)
    ohe = jnp.where(lane_f == ei_col, 1.0, 0.0)
    m1m = jax.lax.dot_general(ohs, gt_ref[0], (((1,), (0,)), ((), ())),
                              precision=hp)
    iou_ref[0] = jnp.sum(m1m * ohe, axis=1, keepdims=True)
    s_f = si_col
    e_f = ei_col
    ctx = (e_f - s_f) * _EXPAND
    s_exp = s_f - ctx
    e_exp = e_f + ctx
    pts = (jax.lax.broadcasted_iota(jnp.int32, (128, _RES), 1)
           .astype(jnp.float32) + 0.5) / _RES
    coords = s_exp + (e_exp - s_exp) * pts
    coords = jnp.clip(coords, 0.0, _TP - 1.0)
    lo = jnp.floor(coords).astype(jnp.int32)
    hi = jnp.minimum(lo + 1, _TP - 1)
    lo_ref[0] = lo
    hi_ref[0] = hi
    w_ref[0] = coords - lo.astype(jnp.float32)


def _proposals(start, end, actionnes, gt_iou_map):
    B, T = start.shape
    sc = jnp.pad(start[:, :, None], ((0, 0), (0, 128 - T), (0, 0)))
    er = jnp.pad(end[:, None, :], ((0, 0), (0, 0), (0, 128 - T)))
    ac = jnp.pad(actionnes[:, :, None], ((0, 0), (0, 128 - T), (0, 0)))
    ar = jnp.pad(actionnes[:, None, :], ((0, 0), (0, 0), (0, 128 - T)))
    gt = jnp.pad(gt_iou_map, ((0, 0), (0, 128 - T), (0, 128 - T)))
    bmap = lambda b: (b, 0, 0)
    return pl.pallas_call(
        _prop_body,
        grid=(B,),
        in_specs=[
            pl.BlockSpec((1, 128, 1), bmap),
            pl.BlockSpec((1, 1, 128), bmap),
            pl.BlockSpec((1, 128, 1), bmap),
            pl.BlockSpec((1, 1, 128), bmap),
            pl.BlockSpec((1, 128, 128), bmap),
        ],
        out_specs=[
            pl.BlockSpec((1, 1, 128), bmap),
            pl.BlockSpec((1, 1, 128), bmap),
            pl.BlockSpec((1, 128, 1), bmap),
            pl.BlockSpec((1, 128, _RES), bmap),
            pl.BlockSpec((1, 128, _RES), bmap),
            pl.BlockSpec((1, 128, _RES), bmap),
        ],
        out_shape=[
            jax.ShapeDtypeStruct((B, 1, 128), jnp.int32),
            jax.ShapeDtypeStruct((B, 1, 128), jnp.int32),
            jax.ShapeDtypeStruct((B, 128, 1), jnp.float32),
            jax.ShapeDtypeStruct((B, 128, _RES), jnp.int32),
            jax.ShapeDtypeStruct((B, 128, _RES), jnp.int32),
            jax.ShapeDtypeStruct((B, 128, _RES), jnp.float32),
        ],
        scratch_shapes=[
            pltpu.VMEM((128, 128), jnp.float32),
        ],
        compiler_params=pltpu.CompilerParams(
            dimension_semantics=("parallel",)),
    )(sc, er, ac, ar, gt)


def _align_body(lo_ref, hi_ref, w_ref, x_ref, out_ref):
    T = x_ref.shape[2]
    lo = jnp.broadcast_to(lo_ref[0, 0], (T, _NL))
    hi = jnp.broadcast_to(hi_ref[0, 0], (T, _NL))
    w = jnp.broadcast_to(w_ref[0, 0], (T, _NL))
    t = jax.lax.broadcasted_iota(jnp.int32, (T, _NL), 0)
    wmat = jnp.where(t == lo, 1.0 - w, 0.0) + jnp.where(t == hi, w, 0.0)
    res = jax.lax.dot_general(x_ref[0], wmat, (((1,), (0,)), ((), ())),
                              preferred_element_type=jnp.float32)
    for i in range(_CH):
        out_ref[i] = res[:, i * _RES:(i + 1) * _RES]


def _align(x, lo_r, hi_r, w_r):
    B, C, T = x.shape
    nch = _TP // _CH
    spec_idx = pl.BlockSpec((1, 1, 1, _NL), lambda b, c: (b, c, 0, 0))
    return pl.pallas_call(
        _align_body,
        grid=(B, nch),
        in_specs=[
            spec_idx, spec_idx, spec_idx,
            pl.BlockSpec((1, C, T), lambda b, c: (b, 0, 0)),
        ],
        out_specs=pl.BlockSpec((_CH, C, _RES), lambda b, c: (b * nch + c, 0, 0)),
        out_shape=jax.ShapeDtypeStruct((B * _TP, C, _RES), jnp.float32),
        compiler_params=pltpu.CompilerParams(
            dimension_semantics=("parallel", "arbitrary")),
    )(lo_r, hi_r, w_r, x)


def kernel(x, start, end, actionnes, gt_iou_map, gt_bbox, num_gt):
    B, C, T = x.shape
    nch = _TP // _CH
    si_o, ei_o, iou_o, lo_o, hi_o, w_o = _proposals(
        start, end, actionnes, gt_iou_map)
    s_i = si_o[:, 0, :_TP].reshape(-1)
    e_i = ei_o[:, 0, :_TP].reshape(-1)
    b_idx = jnp.repeat(jnp.arange(B, dtype=jnp.int32), _TP)
    s_f = s_i.astype(jnp.float32)
    e_f = e_i.astype(jnp.float32)
    anchor_coord = jnp.stack([b_idx.astype(jnp.float32), s_f, e_f], axis=1)
    iou = iou_o[:, :_TP, 0].reshape(-1)
    samp_thr = jnp.mean(iou)
    pos_idx_st_end = (iou > samp_thr).astype(jnp.float32)
    lo_r = lo_o[:, :_TP, :].reshape(B, nch, 1, _NL)
    hi_r = hi_o[:, :_TP, :].reshape(B, nch, 1, _NL)
    w_r = w_o[:, :_TP, :].reshape(B, nch, 1, _NL)
    feat = _align(x, lo_r, hi_r, w_r)
    anchor_num = jnp.full((B,), _TP, dtype=jnp.int32)
    return feat, anchor_coord, anchor_num, samp_thr, pos_idx_st_end


# speedup vs baseline: 1.8863x; 1.5962x over previous
import jax
import jax.numpy as jnp
from jax.experimental import pallas as pl
from jax.experimental.pallas import tpu as pltpu

_RES = 32
_TP = 100
_EXPAND = 0.5
_CH = 20
_NL = _CH * _RES
_NEG = -1e9


_GB = 8


def _prop_body(st_ref, ac_ref, en_ref, ar_ref, gt_ref,
               si_ref, ei_ref, iou_ref, lo_ref, hi_ref, w_ref,
               score_scr):
    hp = jax.lax.Precision.HIGHEST
    sub2 = jax.lax.broadcasted_iota(jnp.int32, (1024, 128), 0)
    lane2 = jax.lax.broadcasted_iota(jnp.int32, (1024, 128), 1)
    sloc2 = jnp.remainder(sub2, 128)
    score = (st_ref[0] * en_ref[0]) * 0.5 * (ac_ref[0] + ar_ref[0])
    valid = (lane2 > sloc2) & (lane2 < _TP)
    score = jnp.where(valid, score, _NEG)
    score_scr[...] = score
    rm_col = jnp.max(score, axis=1, keepdims=True)
    lane2f = lane2.astype(jnp.float32)
    ra_col = jnp.min(jnp.where(score == rm_col, lane2f, 999.0),
                     axis=1, keepdims=True)

    sub128 = jax.lax.broadcasted_iota(jnp.int32, (128, 128), 0)
    lane128 = jax.lax.broadcasted_iota(jnp.int32, (128, 128), 1)
    eye = jnp.where(sub128 == lane128, 1.0, 0.0)
    rm_rows = []
    ra_rows = []
    for b in range(_GB):
        rm_b = rm_col[128 * b:128 * (b + 1), :]
        ra_b = ra_col[128 * b:128 * (b + 1), :]
        rm_rows.append(jax.lax.dot_general(
            rm_b, eye, (((0,), (0,)), ((), ())), precision=hp))
        ra_rows.append(jax.lax.dot_general(
            ra_b, eye, (((0,), (0,)), ((), ())), precision=hp))
    rm0 = jnp.concatenate(rm_rows, axis=0)
    ra0 = jnp.concatenate(ra_rows, axis=0)

    lane8 = jax.lax.broadcasted_iota(jnp.int32, (_GB, 128), 1)
    lane8f = lane8.astype(jnp.float32)
    grp = jax.lax.broadcasted_iota(jnp.int32, (_GB, 1024), 1) // 128
    sub8 = jax.lax.broadcasted_iota(jnp.int32, (_GB, 1024), 0)
    grpeq = grp == sub8
    sloc8f = jnp.remainder(
        jax.lax.broadcasted_iota(jnp.int32, (_GB, 1024), 1), 128
    ).astype(jnp.float32)

    def body(k, carry):
        rm, ra, si_acc, ei_acc = carry
        gmax = jnp.max(rm, axis=1, keepdims=True)
        r_vec = jnp.min(jnp.where(rm == gmax, lane8f, 999.0),
                        axis=1, keepdims=True)
        rmask = lane8f == r_vec
        e_vec = jnp.sum(jnp.where(rmask, ra, 0.0),
                        axis=1, keepdims=True)
        si_acc = jnp.where(lane8 == k, r_vec, si_acc)
        ei_acc = jnp.where(lane8 == k, e_vec, ei_acc)
        rbc = jnp.broadcast_to(r_vec, (_GB, 1024))
        rowsel = jnp.where((sloc8f == rbc) & grpeq, 1.0, 0.0)
        rows = jax.lax.dot_general(
            rowsel, score_scr[...], (((1,), (0,)), ((), ())),
            precision=hp)
        colmask = jnp.where(lane8f == e_vec, 1.0, 0.0)
        rows2 = rows - 4e9 * colmask
        m1 = jnp.max(rows2, axis=1, keepdims=True)
        a1 = jnp.min(jnp.where(rows2 == m1, lane8f, 999.0),
                     axis=1, keepdims=True)
        rm = jnp.where(rmask, m1, rm)
        ra = jnp.where(rmask, a1, ra)
        delta = jax.lax.dot_general(
            rowsel, colmask, (((0,), (0,)), ((), ())),
            precision=hp)
        score_scr[...] = score_scr[...] - 4e9 * delta
        return rm, ra, si_acc, ei_acc

    zf = jnp.zeros((_GB, 128), jnp.float32)
    rm, ra, si_acc, ei_acc = jax.lax.fori_loop(0, _TP, body,
                                               (rm0, ra0, zf, zf))
    si_ref[0] = si_acc.astype(jnp.int32)
    ei_ref[0] = ei_acc.astype(jnp.int32)
    lane128f = lane128.astype(jnp.float32)
    pts = (jax.lax.broadcasted_iota(jnp.int32, (128, _RES), 1)
           .astype(jnp.float32) + 0.5) / _RES
    for b in range(_GB):
        si_b = si_acc[b:b + 1, :]
        ei_b = ei_acc[b:b + 1, :]
        si_col = jax.lax.dot_general(
            eye, si_b, (((1,), (1,)), ((), ())), precision=hp)
        ei_col = jax.lax.dot_general(
            eye, ei_b, (((1,), (1,)), ((), ())), precision=hp)
        ohs = jnp.where(lane128f == si_col, 1.0, 0.0)
        ohe = jnp.where(lane128f == ei_col, 1.0, 0.0)
        gtb = gt_ref[0, 128 * b:128 * (b + 1), :]
        m1m = jax.lax.dot_general(
            ohs, gtb, (((1,), (0,)), ((), ())), precision=hp)
        iou_ref[0, 128 * b:128 * (b + 1), :] = jnp.sum(
            m1m * ohe, axis=1, keepdims=True)
        s_f = si_col
        e_f = ei_col
        ctx = (e_f - s_f) * _EXPAND
        s_exp = s_f - ctx
        e_exp = e_f + ctx
        coords = s_exp + (e_exp - s_exp) * pts
        coords = jnp.clip(coords, 0.0, _TP - 1.0)
        lo = jnp.floor(coords).astype(jnp.int32)
        hi = jnp.minimum(lo + 1, _TP - 1)
        lo_ref[0, 128 * b:128 * (b + 1), :] = lo
        hi_ref[0, 128 * b:128 * (b + 1), :] = hi
        w_ref[0, 128 * b:128 * (b + 1), :] = coords - lo.astype(jnp.float32)


def _proposals(start, end, actionnes, gt_iou_map):
    B, T = start.shape
    ng = B // _GB
    rows = _GB * 128
    stp = jnp.pad(start, ((0, 0), (0, 128 - T)))
    acp = jnp.pad(actionnes, ((0, 0), (0, 128 - T)))
    enp = jnp.pad(end, ((0, 0), (0, 128 - T)))
    st = stp.reshape(ng, rows, 1)
    ac = acp.reshape(ng, rows, 1)
    en = jnp.broadcast_to(enp[:, None, :], (B, 128, 128)).reshape(
        ng, rows, 128)
    ar = jnp.broadcast_to(acp[:, None, :], (B, 128, 128)).reshape(
        ng, rows, 128)
    gt = jnp.pad(gt_iou_map,
                 ((0, 0), (0, 128 - T), (0, 128 - T))).reshape(ng, rows, 128)
    bmap = lambda g: (g, 0, 0)
    return pl.pallas_call(
        _prop_body,
        grid=(ng,),
        in_specs=[
            pl.BlockSpec((1, rows, 1), bmap),
            pl.BlockSpec((1, rows, 1), bmap),
            pl.BlockSpec((1, rows, 128), bmap),
            pl.BlockSpec((1, rows, 128), bmap),
            pl.BlockSpec((1, rows, 128), bmap),
        ],
        out_specs=[
            pl.BlockSpec((1, _GB, 128), bmap),
            pl.BlockSpec((1, _GB, 128), bmap),
            pl.BlockSpec((1, rows, 1), bmap),
            pl.BlockSpec((1, rows, _RES), bmap),
            pl.BlockSpec((1, rows, _RES), bmap),
            pl.BlockSpec((1, rows, _RES), bmap),
        ],
        out_shape=[
            jax.ShapeDtypeStruct((ng, _GB, 128), jnp.int32),
            jax.ShapeDtypeStruct((ng, _GB, 128), jnp.int32),
            jax.ShapeDtypeStruct((ng, rows, 1), jnp.float32),
            jax.ShapeDtypeStruct((ng, rows, _RES), jnp.int32),
            jax.ShapeDtypeStruct((ng, rows, _RES), jnp.int32),
            jax.ShapeDtypeStruct((ng, rows, _RES), jnp.float32),
        ],
        scratch_shapes=[
            pltpu.VMEM((rows, 128), jnp.float32),
        ],
        compiler_params=pltpu.CompilerParams(
            dimension_semantics=("parallel",)),
    )(st, ac, en, ar, gt)


def _align_body(lo_ref, hi_ref, w_ref, x_ref, out_ref):
    T = x_ref.shape[2]
    lo = jnp.broadcast_to(lo_ref[0, 0], (T, _NL))
    hi = jnp.broadcast_to(hi_ref[0, 0], (T, _NL))
    w = jnp.broadcast_to(w_ref[0, 0], (T, _NL))
    t = jax.lax.broadcasted_iota(jnp.int32, (T, _NL), 0)
    wmat = jnp.where(t == lo, 1.0 - w, 0.0) + jnp.where(t == hi, w, 0.0)
    res = jax.lax.dot_general(x_ref[0], wmat, (((1,), (0,)), ((), ())),
                              preferred_element_type=jnp.float32)
    for i in range(_CH):
        out_ref[i] = res[:, i * _RES:(i + 1) * _RES]


def _align(x, lo_r, hi_r, w_r):
    B, C, T = x.shape
    nch = _TP // _CH
    spec_idx = pl.BlockSpec((1, 1, 1, _NL), lambda b, c: (b, c, 0, 0))
    return pl.pallas_call(
        _align_body,
        grid=(B, nch),
        in_specs=[
            spec_idx, spec_idx, spec_idx,
            pl.BlockSpec((1, C, T), lambda b, c: (b, 0, 0)),
        ],
        out_specs=pl.BlockSpec((_CH, C, _RES), lambda b, c: (b * nch + c, 0, 0)),
        out_shape=jax.ShapeDtypeStruct((B * _TP, C, _RES), jnp.float32),
        compiler_params=pltpu.CompilerParams(
            dimension_semantics=("parallel", "arbitrary")),
    )(lo_r, hi_r, w_r, x)


def kernel(x, start, end, actionnes, gt_iou_map, gt_bbox, num_gt):
    B, C, T = x.shape
    nch = _TP // _CH
    si_o, ei_o, iou_o, lo_o, hi_o, w_o = _proposals(
        start, end, actionnes, gt_iou_map)
    s_i = si_o.reshape(B, 128)[:, :_TP].reshape(-1)
    e_i = ei_o.reshape(B, 128)[:, :_TP].reshape(-1)
    b_idx = jnp.repeat(jnp.arange(B, dtype=jnp.int32), _TP)
    s_f = s_i.astype(jnp.float32)
    e_f = e_i.astype(jnp.float32)
    anchor_coord = jnp.stack([b_idx.astype(jnp.float32), s_f, e_f], axis=1)
    iou = iou_o.reshape(B, 128)[:, :_TP].reshape(-1)
    samp_thr = jnp.mean(iou)
    pos_idx_st_end = (iou > samp_thr).astype(jnp.float32)
    lo_r = lo_o.reshape(B, 128, _RES)[:, :_TP, :].reshape(B, nch, 1, _NL)
    hi_r = hi_o.reshape(B, 128, _RES)[:, :_TP, :].reshape(B, nch, 1, _NL)
    w_r = w_o.reshape(B, 128, _RES)[:, :_TP, :].reshape(B, nch, 1, _NL)
    feat = _align(x, lo_r, hi_r, w_r)
    anchor_num = jnp.full((B,), _TP, dtype=jnp.int32)
    return feat, anchor_coord, anchor_num, samp_thr, pos_idx_st_end


# speedup vs baseline: 3.4544x; 1.8313x over previous
import jax
import jax.numpy as jnp
from jax.experimental import pallas as pl
from jax.experimental.pallas import tpu as pltpu

_RES = 32
_TP = 100
_EXPAND = 0.5
_CH = 20
_NL = _CH * _RES
_NEG = -1e9


_GB = 8


def _prop_body(st_ref, ac_ref, en_ref, ar_ref, gt_ref,
               si_ref, ei_ref, iou_ref, lo_ref, hi_ref, w_ref,
               score_scr):
    hp = jax.lax.Precision.HIGHEST
    sub2 = jax.lax.broadcasted_iota(jnp.int32, (1024, 128), 0)
    lane2 = jax.lax.broadcasted_iota(jnp.int32, (1024, 128), 1)
    sloc2 = jnp.remainder(sub2, 128)
    score = (st_ref[0] * en_ref[0]) * 0.5 * (ac_ref[0] + ar_ref[0])
    valid = (lane2 > sloc2) & (lane2 < _TP)
    score = jnp.where(valid, score, _NEG)
    score_scr[...] = score
    rm_col = jnp.max(score, axis=1, keepdims=True)
    lane2f = lane2.astype(jnp.float32)
    ra_col = jnp.min(jnp.where(score == rm_col, lane2f, 999.0),
                     axis=1, keepdims=True)

    sub128 = jax.lax.broadcasted_iota(jnp.int32, (128, 128), 0)
    lane128 = jax.lax.broadcasted_iota(jnp.int32, (128, 128), 1)
    eye = jnp.where(sub128 == lane128, 1.0, 0.0)
    rm_rows = []
    ra_rows = []
    for b in range(_GB):
        rm_b = rm_col[128 * b:128 * (b + 1), :]
        ra_b = ra_col[128 * b:128 * (b + 1), :]
        rm_rows.append(jax.lax.dot_general(
            rm_b, eye, (((0,), (0,)), ((), ())), precision=hp))
        ra_rows.append(jax.lax.dot_general(
            ra_b, eye, (((0,), (0,)), ((), ())), precision=hp))
    rm0 = jnp.concatenate(rm_rows, axis=0)
    ra0 = jnp.concatenate(ra_rows, axis=0)

    lane8 = jax.lax.broadcasted_iota(jnp.int32, (_GB, 128), 1)
    lane8f = lane8.astype(jnp.float32)
    grp = jax.lax.broadcasted_iota(jnp.int32, (_GB, 1024), 1) // 128
    sub8 = jax.lax.broadcasted_iota(jnp.int32, (_GB, 1024), 0)
    grpeq = grp == sub8
    sloc8f = jnp.remainder(
        jax.lax.broadcasted_iota(jnp.int32, (_GB, 1024), 1), 128
    ).astype(jnp.float32)

    def body(k, carry):
        rm, ra, si_acc, ei_acc = carry
        gmax = jnp.max(rm, axis=1, keepdims=True)
        r_vec = jnp.min(jnp.where(rm == gmax, lane8f, 999.0),
                        axis=1, keepdims=True)
        rmask = lane8f == r_vec
        e_vec = jnp.sum(jnp.where(rmask, ra, 0.0),
                        axis=1, keepdims=True)
        si_acc = jnp.where(lane8 == k, r_vec, si_acc)
        ei_acc = jnp.where(lane8 == k, e_vec, ei_acc)
        rbc = jnp.broadcast_to(r_vec, (_GB, 1024))
        rowsel = jnp.where((sloc8f == rbc) & grpeq, 1.0, 0.0)
        rows = jax.lax.dot_general(
            rowsel, score_scr[...], (((1,), (0,)), ((), ())),
            precision=hp)
        colmask = jnp.where(lane8f == e_vec, 1.0, 0.0)
        rows2 = rows - 4e9 * colmask
        m1 = jnp.max(rows2, axis=1, keepdims=True)
        a1 = jnp.min(jnp.where(rows2 == m1, lane8f, 999.0),
                     axis=1, keepdims=True)
        rm = jnp.where(rmask, m1, rm)
        ra = jnp.where(rmask, a1, ra)
        delta = jax.lax.dot_general(
            rowsel, colmask, (((0,), (0,)), ((), ())),
            precision=hp)
        score_scr[...] = score_scr[...] - 4e9 * delta
        return rm, ra, si_acc, ei_acc

    zf = jnp.zeros((_GB, 128), jnp.float32)
    rm, ra, si_acc, ei_acc = jax.lax.fori_loop(0, _TP, body,
                                               (rm0, ra0, zf, zf))
    si_ref[0] = si_acc.astype(jnp.int32)
    ei_ref[0] = ei_acc.astype(jnp.int32)
    lane128f = lane128.astype(jnp.float32)
    pts = (jax.lax.broadcasted_iota(jnp.int32, (128, _RES), 1)
           .astype(jnp.float32) + 0.5) / _RES
    for b in range(_GB):
        si_b = si_acc[b:b + 1, :]
        ei_b = ei_acc[b:b + 1, :]
        si_col = jax.lax.dot_general(
            eye, si_b, (((1,), (1,)), ((), ())), precision=hp)
        ei_col = jax.lax.dot_general(
            eye, ei_b, (((1,), (1,)), ((), ())), precision=hp)
        ohs = jnp.where(lane128f == si_col, 1.0, 0.0)
        ohe = jnp.where(lane128f == ei_col, 1.0, 0.0)
        gtb = gt_ref[0, 128 * b:128 * (b + 1), :]
        m1m = jax.lax.dot_general(
            ohs, gtb, (((1,), (0,)), ((), ())), precision=hp)
        iou_ref[0, 128 * b:128 * (b + 1), :] = jnp.sum(
            m1m * ohe, axis=1, keepdims=True)
        s_f = si_col
        e_f = ei_col
        ctx = (e_f - s_f) * _EXPAND
        s_exp = s_f - ctx
        e_exp = e_f + ctx
        coords = s_exp + (e_exp - s_exp) * pts
        coords = jnp.clip(coords, 0.0, _TP - 1.0)
        lo = jnp.floor(coords).astype(jnp.int32)
        hi = jnp.minimum(lo + 1, _TP - 1)
        lo_ref[0, 128 * b:128 * (b + 1), :] = lo
        hi_ref[0, 128 * b:128 * (b + 1), :] = hi
        w_ref[0, 128 * b:128 * (b + 1), :] = coords - lo.astype(jnp.float32)


def _proposals(start, end, actionnes, gt_iou_map):
    B, T = start.shape
    ng = B // _GB
    rows = _GB * 128
    stp = jnp.pad(start, ((0, 0), (0, 128 - T)))
    acp = jnp.pad(actionnes, ((0, 0), (0, 128 - T)))
    enp = jnp.pad(end, ((0, 0), (0, 128 - T)))
    st = stp.reshape(ng, rows, 1)
    ac = acp.reshape(ng, rows, 1)
    en = jnp.broadcast_to(enp[:, None, :], (B, 128, 128)).reshape(
        ng, rows, 128)
    ar = jnp.broadcast_to(acp[:, None, :], (B, 128, 128)).reshape(
        ng, rows, 128)
    gt = jnp.pad(gt_iou_map,
                 ((0, 0), (0, 128 - T), (0, 128 - T))).reshape(ng, rows, 128)
    bmap = lambda g: (g, 0, 0)
    return pl.pallas_call(
        _prop_body,
        grid=(ng,),
        in_specs=[
            pl.BlockSpec((1, rows, 1), bmap),
            pl.BlockSpec((1, rows, 1), bmap),
            pl.BlockSpec((1, rows, 128), bmap),
            pl.BlockSpec((1, rows, 128), bmap),
            pl.BlockSpec((1, rows, 128), bmap),
        ],
        out_specs=[
            pl.BlockSpec((1, _GB, 128), bmap),
            pl.BlockSpec((1, _GB, 128), bmap),
            pl.BlockSpec((1, rows, 1), bmap),
            pl.BlockSpec((1, rows, _RES), bmap),
            pl.BlockSpec((1, rows, _RES), bmap),
            pl.BlockSpec((1, rows, _RES), bmap),
        ],
        out_shape=[
            jax.ShapeDtypeStruct((ng, _GB, 128), jnp.int32),
            jax.ShapeDtypeStruct((ng, _GB, 128), jnp.int32),
            jax.ShapeDtypeStruct((ng, rows, 1), jnp.float32),
            jax.ShapeDtypeStruct((ng, rows, _RES), jnp.int32),
            jax.ShapeDtypeStruct((ng, rows, _RES), jnp.int32),
            jax.ShapeDtypeStruct((ng, rows, _RES), jnp.float32),
        ],
        scratch_shapes=[
            pltpu.VMEM((rows, 128), jnp.float32),
        ],
        compiler_params=pltpu.CompilerParams(
            dimension_semantics=("parallel",)),
    )(st, ac, en, ar, gt)


def kernel(x, start, end, actionnes, gt_iou_map, gt_bbox, num_gt):
    B, C, T = x.shape
    nch = _TP // _CH
    si_o, ei_o, iou_o, lo_o, hi_o, w_o = _proposals(
        start, end, actionnes, gt_iou_map)
    s_i = si_o.reshape(B, 128)[:, :_TP].reshape(-1)
    e_i = ei_o.reshape(B, 128)[:, :_TP].reshape(-1)
    b_idx = jnp.repeat(jnp.arange(B, dtype=jnp.int32), _TP)
    s_f = s_i.astype(jnp.float32)
    e_f = e_i.astype(jnp.float32)
    anchor_coord = jnp.stack([b_idx.astype(jnp.float32), s_f, e_f], axis=1)
    iou = iou_o.reshape(B, 128)[:, :_TP].reshape(-1)
    samp_thr = jnp.mean(iou)
    pos_idx_st_end = (iou > samp_thr).astype(jnp.float32)
    lo_r = lo_o.reshape(B, 128, _RES)[:, :_TP, :].reshape(B, nch, 1, _NL)
    hi_r = hi_o.reshape(B, 128, _RES)[:, :_TP, :].reshape(B, nch, 1, _NL)
    w_r = w_o.reshape(B, 128, _RES)[:, :_TP, :].reshape(B, nch, 1, _NL)
    a2 = _align2(x, lo_r, hi_r, w_r)
    feat = a2.reshape(B, C, _TP, _RES).transpose(0, 2, 1, 3).reshape(
        B * _TP, C, _RES)
    anchor_num = jnp.full((B,), _TP, dtype=jnp.int32)
    return feat, anchor_coord, anchor_num, samp_thr, pos_idx_st_end


def _align2_body(lo_ref, hi_ref, w_ref, x_ref, out_ref):
    T = x_ref.shape[2]
    lo = jnp.broadcast_to(lo_ref[0, 0], (T, _NL))
    hi = jnp.broadcast_to(hi_ref[0, 0], (T, _NL))
    w = jnp.broadcast_to(w_ref[0, 0], (T, _NL))
    t = jax.lax.broadcasted_iota(jnp.int32, (T, _NL), 0)
    wmat = jnp.where(t == lo, 1.0 - w, 0.0) + jnp.where(t == hi, w, 0.0)
    res = jax.lax.dot_general(x_ref[0], wmat, (((1,), (0,)), ((), ())),
                              preferred_element_type=jnp.float32)
    out_ref[0] = res


def _align2(x, lo_r, hi_r, w_r):
    B, C, T = x.shape
    nch = _TP // _CH
    spec_idx = pl.BlockSpec((1, 1, 1, _NL), lambda b, c: (b, c, 0, 0))
    return pl.pallas_call(
        _align2_body,
        grid=(B, nch),
        in_specs=[
            spec_idx, spec_idx, spec_idx,
            pl.BlockSpec((1, C, T), lambda b, c: (b, 0, 0)),
        ],
        out_specs=pl.BlockSpec((1, C, _NL), lambda b, c: (b, 0, c)),
        out_shape=jax.ShapeDtypeStruct((B, C, _TP * _RES), jnp.float32),
        compiler_params=pltpu.CompilerParams(
            dimension_semantics=("parallel", "arbitrary")),
    )(lo_r, hi_r, w_r, x)
